# baseline (speedup 1.0000x reference)
import functools

import jax
import jax.numpy as jnp
from jax import lax
from jax.experimental import pallas as pl
from jax.experimental.pallas import tpu as pltpu
from jax.experimental.pallas import tpu_sc as plsc

NUM_FEATURES = 26
BATCH = 4096
AVG_LEN = 20
PER_FEATURE = BATCH * AVG_LEN
TOTAL = NUM_FEATURES * PER_FEATURE
ZCH_SIZE = 800000
MAX_OUTPUT_ID = 1000000
RESIDUAL = MAX_OUTPUT_ID - ZCH_SIZE

L = 16
NC, NS = 2, 16
NW = NC * NS
PER_TILE = TOTAL // NW
ROWS = ZCH_SIZE // L
SPAD = ROWS + 8
CHUNK = 1280
SUB = 128
NSUB = CHUNK // SUB
VB = 8
GROUP = 2
GIDS = GROUP * CHUNK
NGROUP = PER_TILE // GIDS
BIG = 2**31 - 1


def _build_kernel():
    mesh = plsc.VectorSubcoreMesh(core_axis_name="c", subcore_axis_name="s")

    @functools.partial(
        pl.kernel,
        mesh=mesh,
        compiler_params=pltpu.CompilerParams(
            use_tc_tiling_on_sc=False, needs_layout_passes=False),
        out_type=jax.ShapeDtypeStruct((TOTAL * 2,), jnp.int32),
        scratch_types=[
            pltpu.VMEM((SPAD,), jnp.int32),
            pltpu.VMEM((GIDS * 2,), jnp.int32),
            pltpu.VMEM((CHUNK,), jnp.int32),
            pltpu.VMEM((CHUNK,), jnp.int32),
            pltpu.VMEM((CHUNK,), jnp.int32),
            pltpu.VMEM((CHUNK,), jnp.int32),
            pltpu.VMEM((CHUNK, L), jnp.int32),
            pltpu.VMEM((CHUNK, L), jnp.int32),
            pltpu.VMEM((GIDS * 2,), jnp.int32),
            pltpu.SemaphoreType.DMA,
            pltpu.SemaphoreType.DMA,
        ],
    )
    def remap(ids_hbm, s_hbm, r_hbm, out_hbm,
              s_buf, gids, j0A, j0B, jmA, jmB, rowsA, rowsB, outg,
              semA, semB):
        wid = lax.axis_index("s") * NC + lax.axis_index("c")
        tsel = (wid >= NW // 2).astype(jnp.int32)
        roff = tsel * ROWS
        iota = lax.iota(jnp.int32, L)

        pltpu.sync_copy(s_hbm.at[tsel], s_buf)

        zeros16 = jnp.zeros((L,), jnp.int32)

        def zinit(v, _):
            outg[pl.ds(v * L, L)] = zeros16
            return jnp.int32(0)

        lax.fori_loop(jnp.int32(0), jnp.int32(GIDS * 2 // L), zinit,
                      jnp.int32(0))

        def pass1(r, j0_ref, jm_ref):
            def p1(u, _):
                vb = u * jnp.int32(VB)
                off = jnp.int32(r * CHUNK)
                xs = [plsc.load_gather(
                    gids, [(off + (vb + t) * L + iota) * 2])
                      for t in range(VB)]
                los = [jnp.zeros((L,), jnp.int32) for _ in range(VB)]
                his = [jnp.full((L,), ROWS, jnp.int32) for _ in range(VB)]
                for _step in range(17):
                    mids = [(los[t] + his[t]) >> 1 for t in range(VB)]
                    svs = [plsc.load_gather(s_buf, [mids[t]])
                           for t in range(VB)]
                    for t in range(VB):
                        pred = svs[t] < xs[t]
                        los[t] = jnp.where(pred, mids[t] + 1, los[t])
                        his[t] = jnp.where(pred, his[t], mids[t])
                for t in range(VB):
                    j0_ref[pl.ds((vb + t) * L, L)] = los[t]
                    jm_ref[pl.ds((vb + t) * L, L)] = (
                        jnp.maximum(los[t] - 1, 0) + roff)
                return jnp.int32(0)

            lax.fori_loop(jnp.int32(0), jnp.int32(CHUNK // L // VB), p1,
                          jnp.int32(0))

        def fire(jm_ref, rows_ref, sem):
            return [pltpu.async_copy(
                r_hbm.at[jm_ref.at[pl.ds(jnp.int32(s * SUB), SUB)]],
                rows_ref.at[pl.ds(jnp.int32(s * SUB), SUB)],
                sem) for s in range(NSUB)]

        def pass2(r, j0_ref, rows_ref):
            def p2(v, _):
                off = jnp.int32(r * CHUNK)
                x = plsc.load_gather(gids, [(off + v * L + iota) * 2])
                j0 = j0_ref[pl.ds(v * L, L)]
                rowi = v * L + iota
                cnt = jnp.zeros((L,), jnp.int32)
                for k in range(L):
                    col = plsc.load_gather(
                        rows_ref, [rowi, jnp.full((L,), k, jnp.int32)])
                    cnt = cnt + (col < x).astype(jnp.int32)
                idx = jnp.maximum(j0 - 1, 0) * L + cnt
                va = plsc.load_gather(rows_ref,
                                      [rowi, jnp.minimum(cnt, L - 1)])
                vb_ = plsc.load_gather(s_buf, [jnp.minimum(j0, ROWS)])
                mval = jnp.where(cnt >= L, vb_, va)
                fallback = ZCH_SIZE + lax.rem(x, jnp.int32(RESIDUAL))
                res = jnp.where(mval == x,
                                jnp.minimum(idx, ZCH_SIZE - 1), fallback)
                plsc.store_scatter(outg, [(off + v * L + iota) * 2], res)
                return jnp.int32(0)

            lax.fori_loop(jnp.int32(0), jnp.int32(CHUNK // L), p2,
                          jnp.int32(0))

        def group_body(g, carry):
            base = wid * jnp.int32(PER_TILE) + g * jnp.int32(GIDS)
            pltpu.sync_copy(ids_hbm.at[pl.ds(base * 2, GIDS * 2)], gids)

            pass1(0, j0A, jmA)
            cA = fire(jmA, rowsA, semA)
            pass1(1, j0B, jmB)
            cB = fire(jmB, rowsB, semB)
            for cp in cA:
                cp.wait()
            pass2(0, j0A, rowsA)
            for cp in cB:
                cp.wait()
            pass2(1, j0B, rowsB)

            pltpu.sync_copy(outg, out_hbm.at[pl.ds(base * 2, GIDS * 2)])
            return carry

        lax.fori_loop(jnp.int32(0), jnp.int32(NGROUP), group_body,
                      jnp.int32(0))

    return remap


def kernel(values, lengths, weights, zch_t0, zch_t1):
    z0 = zch_t0.astype(jnp.int32)
    z1 = zch_t1.astype(jnp.int32)
    r_tab = jnp.concatenate(
        [z0.reshape(ROWS, L), z1.reshape(ROWS, L)], axis=0)
    pad = jnp.full((SPAD - ROWS,), BIG, jnp.int32)
    s_tab = jnp.stack([jnp.concatenate([z0[::L], pad]),
                       jnp.concatenate([z1[::L], pad])])
    ids_pairs = lax.bitcast_convert_type(values, jnp.int32).reshape(TOTAL * 2)
    out_pairs = _build_kernel()(ids_pairs, s_tab, r_tab)
    out64 = lax.bitcast_convert_type(out_pairs.reshape(TOTAL, 2), values.dtype)
    return out64, lengths, weights


# speedup vs baseline: 6.1718x; 6.1718x over previous
import functools

import jax
import jax.numpy as jnp
from jax import lax
from jax.experimental import pallas as pl
from jax.experimental.pallas import tpu as pltpu
from jax.experimental.pallas import tpu_sc as plsc

NUM_FEATURES = 26
BATCH = 4096
AVG_LEN = 20
PER_FEATURE = BATCH * AVG_LEN
TOTAL = NUM_FEATURES * PER_FEATURE
ZCH_SIZE = 800000
MAX_OUTPUT_ID = 1000000
RESIDUAL = MAX_OUTPUT_ID - ZCH_SIZE

L = 16
NC, NS = 2, 16
NW = NC * NS
PER_TILE = TOTAL // NW
ROWS = ZCH_SIZE // L
SPAD = ROWS + 8
CHUNK = 1664
SUB = 128
NSUB = CHUNK // SUB
VB = 8
GROUP = 4
GIDS = GROUP * CHUNK
NGROUP = PER_TILE // GIDS
BIG = 2**31 - 1


def _build_kernel():
    mesh = plsc.VectorSubcoreMesh(core_axis_name="c", subcore_axis_name="s")

    @functools.partial(
        pl.kernel,
        mesh=mesh,
        compiler_params=pltpu.CompilerParams(
            use_tc_tiling_on_sc=False, needs_layout_passes=False),
        out_type=jax.ShapeDtypeStruct((TOTAL,), jnp.int32),
        scratch_types=[
            pltpu.VMEM((SPAD,), jnp.int32),
            pltpu.VMEM((GIDS,), jnp.int32),
            pltpu.VMEM((CHUNK,), jnp.int32),
            pltpu.VMEM((CHUNK,), jnp.int32),
            pltpu.VMEM((CHUNK,), jnp.int32),
            pltpu.VMEM((CHUNK,), jnp.int32),
            pltpu.VMEM((CHUNK, L), jnp.int32),
            pltpu.VMEM((CHUNK, L), jnp.int32),
            pltpu.VMEM((GIDS,), jnp.int32),
            pltpu.SemaphoreType.DMA,
            pltpu.SemaphoreType.DMA,
        ],
    )
    def remap(ids_hbm, s_hbm, r_hbm, out_hbm,
              s_buf, gids, j0A, j0B, jmA, jmB, rowsA, rowsB, outg,
              semA, semB):
        wid = lax.axis_index("s") * NC + lax.axis_index("c")
        tsel = (wid >= NW // 2).astype(jnp.int32)
        roff = tsel * ROWS
        iota = lax.iota(jnp.int32, L)

        pltpu.sync_copy(s_hbm.at[tsel], s_buf)

        def pass1(r, j0_ref, jm_ref):
            def p1(u, _):
                vb = u * jnp.int32(VB)
                off = jnp.int32(r * CHUNK)
                xs = [gids[pl.ds(off + (vb + t) * L, L)] for t in range(VB)]
                los = [jnp.zeros((L,), jnp.int32) for _ in range(VB)]
                his = [jnp.full((L,), ROWS, jnp.int32) for _ in range(VB)]
                for _step in range(17):
                    mids = [(los[t] + his[t]) >> 1 for t in range(VB)]
                    svs = [plsc.load_gather(s_buf, [mids[t]])
                           for t in range(VB)]
                    for t in range(VB):
                        pred = svs[t] < xs[t]
                        los[t] = jnp.where(pred, mids[t] + 1, los[t])
                        his[t] = jnp.where(pred, his[t], mids[t])
                for t in range(VB):
                    j0_ref[pl.ds((vb + t) * L, L)] = los[t]
                    jm_ref[pl.ds((vb + t) * L, L)] = (
                        jnp.maximum(los[t] - 1, 0) + roff)
                return jnp.int32(0)

            lax.fori_loop(jnp.int32(0), jnp.int32(CHUNK // L // VB), p1,
                          jnp.int32(0))

        def fire(jm_ref, rows_ref, sem):
            return [pltpu.async_copy(
                r_hbm.at[jm_ref.at[pl.ds(jnp.int32(s * SUB), SUB)]],
                rows_ref.at[pl.ds(jnp.int32(s * SUB), SUB)],
                sem) for s in range(NSUB)]

        def pass2(r, j0_ref, rows_ref):
            def p2(v, _):
                off = jnp.int32(r * CHUNK)
                x = gids[pl.ds(off + v * L, L)]
                j0 = j0_ref[pl.ds(v * L, L)]
                rowi = v * L + iota
                cnt = jnp.zeros((L,), jnp.int32)
                for k in range(L):
                    col = plsc.load_gather(
                        rows_ref, [rowi, jnp.full((L,), k, jnp.int32)])
                    cnt = cnt + (col < x).astype(jnp.int32)
                idx = jnp.maximum(j0 - 1, 0) * L + cnt
                va = plsc.load_gather(rows_ref,
                                      [rowi, jnp.minimum(cnt, L - 1)])
                vb_ = plsc.load_gather(s_buf, [jnp.minimum(j0, ROWS)])
                mval = jnp.where(cnt >= L, vb_, va)
                fallback = ZCH_SIZE + lax.rem(x, jnp.int32(RESIDUAL))
                res = jnp.where(mval == x,
                                jnp.minimum(idx, ZCH_SIZE - 1), fallback)
                outg[pl.ds(off + v * L, L)] = res
                return jnp.int32(0)

            lax.fori_loop(jnp.int32(0), jnp.int32(CHUNK // L), p2,
                          jnp.int32(0))

        def group_body(g, carry):
            base = wid * jnp.int32(PER_TILE) + g * jnp.int32(GIDS)
            pltpu.sync_copy(ids_hbm.at[pl.ds(base, GIDS)], gids)

            pass1(0, j0A, jmA)
            cA = fire(jmA, rowsA, semA)
            pass1(1, j0B, jmB)
            cB = fire(jmB, rowsB, semB)
            for cp in cA:
                cp.wait()
            pass2(0, j0A, rowsA)
            pass1(2, j0A, jmA)
            cA = fire(jmA, rowsA, semA)
            for cp in cB:
                cp.wait()
            pass2(1, j0B, rowsB)
            pass1(3, j0B, jmB)
            cB = fire(jmB, rowsB, semB)
            for cp in cA:
                cp.wait()
            pass2(2, j0A, rowsA)
            for cp in cB:
                cp.wait()
            pass2(3, j0B, rowsB)

            pltpu.sync_copy(outg, out_hbm.at[pl.ds(base, GIDS)])
            return carry

        lax.fori_loop(jnp.int32(0), jnp.int32(NGROUP), group_body,
                      jnp.int32(0))

    return remap


def kernel(values, lengths, weights, zch_t0, zch_t1):
    z0 = zch_t0.astype(jnp.int32)
    z1 = zch_t1.astype(jnp.int32)
    r_tab = jnp.concatenate(
        [z0.reshape(ROWS, L), z1.reshape(ROWS, L)], axis=0)
    pad = jnp.full((SPAD - ROWS,), BIG, jnp.int32)
    s_tab = jnp.stack([jnp.concatenate([z0[::L], pad]),
                       jnp.concatenate([z1[::L], pad])])
    ids32 = values.astype(jnp.int32)
    out32 = _build_kernel()(ids32, s_tab, r_tab)
    return out32.astype(values.dtype), lengths, weights


# speedup vs baseline: 7.3051x; 1.1836x over previous
import functools

import jax
import jax.numpy as jnp
from jax import lax
from jax.experimental import pallas as pl
from jax.experimental.pallas import tpu as pltpu
from jax.experimental.pallas import tpu_sc as plsc

NUM_FEATURES = 26
BATCH = 4096
AVG_LEN = 20
PER_FEATURE = BATCH * AVG_LEN
TOTAL = NUM_FEATURES * PER_FEATURE
ZCH_SIZE = 800000
MAX_OUTPUT_ID = 1000000
RESIDUAL = MAX_OUTPUT_ID - ZCH_SIZE

L = 16
NC, NS = 2, 16
NW = NC * NS
HALF = TOTAL // 2
PER_TILE = HALF // NW
ROWS = ZCH_SIZE // L
SPAD = ROWS + 8
CHUNK = 1664
SUB = 128
NSUB = CHUNK // SUB
VB = 8
GROUP = 4
GIDS = GROUP * CHUNK
NGROUP = PER_TILE // GIDS
BIG = 2**31 - 1


def _build_kernel():
    mesh = plsc.VectorSubcoreMesh(core_axis_name="c", subcore_axis_name="s")

    @functools.partial(
        pl.kernel,
        mesh=mesh,
        compiler_params=pltpu.CompilerParams(
            use_tc_tiling_on_sc=False, needs_layout_passes=False),
        out_type=jax.ShapeDtypeStruct((HALF,), jnp.int32),
        scratch_types=[
            pltpu.VMEM((SPAD,), jnp.int32),
            pltpu.VMEM((GIDS,), jnp.int32),
            pltpu.VMEM((CHUNK,), jnp.int32),
            pltpu.VMEM((CHUNK,), jnp.int32),
            pltpu.VMEM((CHUNK,), jnp.int32),
            pltpu.VMEM((CHUNK,), jnp.int32),
            pltpu.VMEM((CHUNK, L), jnp.int32),
            pltpu.VMEM((CHUNK, L), jnp.int32),
            pltpu.VMEM((GIDS,), jnp.int32),
            pltpu.SemaphoreType.DMA,
            pltpu.SemaphoreType.DMA,
        ],
    )
    def remap(ids_hbm, s_hbm, r_hbm, out_hbm,
              s_buf, gids, j0A, j0B, jmA, jmB, rowsA, rowsB, outg,
              semA, semB):
        wid = lax.axis_index("s") * NC + lax.axis_index("c")
        iota = lax.iota(jnp.int32, L)

        pltpu.sync_copy(s_hbm, s_buf)

        def pass1(r, j0_ref, jm_ref):
            def p1(u, _):
                vb = u * jnp.int32(VB)
                off = jnp.int32(r * CHUNK)
                xs = [gids[pl.ds(off + (vb + t) * L, L)] for t in range(VB)]
                los = [jnp.zeros((L,), jnp.int32) for _ in range(VB)]
                his = [jnp.full((L,), ROWS, jnp.int32) for _ in range(VB)]
                for _step in range(17):
                    mids = [(los[t] + his[t]) >> 1 for t in range(VB)]
                    svs = [plsc.load_gather(s_buf, [mids[t]])
                           for t in range(VB)]
                    for t in range(VB):
                        pred = svs[t] < xs[t]
                        los[t] = jnp.where(pred, mids[t] + 1, los[t])
                        his[t] = jnp.where(pred, his[t], mids[t])
                for t in range(VB):
                    j0_ref[pl.ds((vb + t) * L, L)] = los[t]
                    jm_ref[pl.ds((vb + t) * L, L)] = (
                        jnp.maximum(los[t] - 1, 0))
                return jnp.int32(0)

            lax.fori_loop(jnp.int32(0), jnp.int32(CHUNK // L // VB), p1,
                          jnp.int32(0))

        def fire(jm_ref, rows_ref, sem):
            return [pltpu.async_copy(
                r_hbm.at[jm_ref.at[pl.ds(jnp.int32(s * SUB), SUB)]],
                rows_ref.at[pl.ds(jnp.int32(s * SUB), SUB)],
                sem) for s in range(NSUB)]

        def pass2(r, j0_ref, rows_ref):
            def p2(v, _):
                off = jnp.int32(r * CHUNK)
                x = gids[pl.ds(off + v * L, L)]
                j0 = j0_ref[pl.ds(v * L, L)]
                rowi = v * L + iota
                cnt = jnp.zeros((L,), jnp.int32)
                for k in range(L):
                    col = plsc.load_gather(
                        rows_ref, [rowi, jnp.full((L,), k, jnp.int32)])
                    cnt = cnt + (col < x).astype(jnp.int32)
                idx = jnp.maximum(j0 - 1, 0) * L + cnt
                va = plsc.load_gather(rows_ref,
                                      [rowi, jnp.minimum(cnt, L - 1)])
                vb_ = plsc.load_gather(s_buf, [jnp.minimum(j0, ROWS)])
                mval = jnp.where(cnt >= L, vb_, va)
                fallback = ZCH_SIZE + lax.rem(x, jnp.int32(RESIDUAL))
                res = jnp.where(mval == x,
                                jnp.minimum(idx, ZCH_SIZE - 1), fallback)
                outg[pl.ds(off + v * L, L)] = res
                return jnp.int32(0)

            lax.fori_loop(jnp.int32(0), jnp.int32(CHUNK // L), p2,
                          jnp.int32(0))

        def group_body(g, carry):
            base = wid * jnp.int32(PER_TILE) + g * jnp.int32(GIDS)
            pltpu.sync_copy(ids_hbm.at[pl.ds(base, GIDS)], gids)

            pass1(0, j0A, jmA)
            cA = fire(jmA, rowsA, semA)
            pass1(1, j0B, jmB)
            cB = fire(jmB, rowsB, semB)
            for cp in cA:
                cp.wait()
            pass2(0, j0A, rowsA)
            pass1(2, j0A, jmA)
            cA = fire(jmA, rowsA, semA)
            for cp in cB:
                cp.wait()
            pass2(1, j0B, rowsB)
            pass1(3, j0B, jmB)
            cB = fire(jmB, rowsB, semB)
            for cp in cA:
                cp.wait()
            pass2(2, j0A, rowsA)
            for cp in cB:
                cp.wait()
            pass2(3, j0B, rowsB)

            pltpu.sync_copy(outg, out_hbm.at[pl.ds(base, GIDS)])
            return carry

        lax.fori_loop(jnp.int32(0), jnp.int32(NGROUP), group_body,
                      jnp.int32(0))

    return remap


def kernel(values, lengths, weights, zch_t0, zch_t1):
    k = _build_kernel()
    pad = jnp.full((SPAD - ROWS,), BIG, jnp.int32)
    z0 = zch_t0.astype(jnp.int32)
    s0 = jnp.concatenate([z0[::L], pad])
    o0 = k(values[:HALF].astype(jnp.int32), s0, z0.reshape(ROWS, L))
    z1 = zch_t1.astype(jnp.int32)
    s1 = jnp.concatenate([z1[::L], pad])
    o1 = k(values[HALF:].astype(jnp.int32), s1, z1.reshape(ROWS, L))
    out = jnp.concatenate([o0.astype(values.dtype), o1.astype(values.dtype)])
    return out, lengths, weights


# speedup vs baseline: 8.6699x; 1.1868x over previous
import functools

import jax
import jax.numpy as jnp
from jax import lax
from jax.experimental import pallas as pl
from jax.experimental.pallas import tpu as pltpu
from jax.experimental.pallas import tpu_sc as plsc

NUM_FEATURES = 26
BATCH = 4096
AVG_LEN = 20
PER_FEATURE = BATCH * AVG_LEN
TOTAL = NUM_FEATURES * PER_FEATURE
ZCH_SIZE = 800000
MAX_OUTPUT_ID = 1000000
RESIDUAL = MAX_OUTPUT_ID - ZCH_SIZE

L = 16
NC, NS = 2, 16
NW = NC * NS
HALF = TOTAL // 2
PER_TILE = HALF // NW
ROWS = ZCH_SIZE // L
SPAD = ROWS + 8
CHUNK = 1664
SUB = 128
NSUB = CHUNK // SUB
VB = 8
VB2 = 4
GROUP = 4
GIDS = GROUP * CHUNK
NGROUP = PER_TILE // GIDS
BIG = 2**31 - 1


def _build_kernel():
    mesh = plsc.VectorSubcoreMesh(core_axis_name="c", subcore_axis_name="s")

    @functools.partial(
        pl.kernel,
        mesh=mesh,
        compiler_params=pltpu.CompilerParams(
            use_tc_tiling_on_sc=False, needs_layout_passes=False),
        out_type=jax.ShapeDtypeStruct((HALF,), jnp.int32),
        scratch_types=[
            pltpu.VMEM((SPAD,), jnp.int32),
            pltpu.VMEM((GIDS,), jnp.int32),
            pltpu.VMEM((CHUNK,), jnp.int32),
            pltpu.VMEM((CHUNK,), jnp.int32),
            pltpu.VMEM((CHUNK,), jnp.int32),
            pltpu.VMEM((CHUNK,), jnp.int32),
            pltpu.VMEM((CHUNK, L), jnp.int32),
            pltpu.VMEM((CHUNK, L), jnp.int32),
            pltpu.VMEM((GIDS,), jnp.int32),
            pltpu.SemaphoreType.DMA,
            pltpu.SemaphoreType.DMA,
        ],
    )
    def remap(ids_hbm, s_hbm, r_hbm, out_hbm,
              s_buf, gids, j0A, j0B, jmA, jmB, rowsA, rowsB, outg,
              semA, semB):
        wid = lax.axis_index("s") * NC + lax.axis_index("c")
        iota = lax.iota(jnp.int32, L)

        pltpu.sync_copy(s_hbm, s_buf)

        def pass1(r, j0_ref, jm_ref):
            def p1(u, _):
                vb = u * jnp.int32(VB)
                off = jnp.int32(r * CHUNK)
                xs = [gids[pl.ds(off + (vb + t) * L, L)] for t in range(VB)]
                los = [jnp.zeros((L,), jnp.int32) for _ in range(VB)]
                his = [jnp.full((L,), ROWS, jnp.int32) for _ in range(VB)]
                for _step in range(17):
                    mids = [(los[t] + his[t]) >> 1 for t in range(VB)]
                    svs = [plsc.load_gather(s_buf, [mids[t]])
                           for t in range(VB)]
                    for t in range(VB):
                        pred = svs[t] < xs[t]
                        los[t] = jnp.where(pred, mids[t] + 1, los[t])
                        his[t] = jnp.where(pred, his[t], mids[t])
                for t in range(VB):
                    j0_ref[pl.ds((vb + t) * L, L)] = los[t]
                    jm_ref[pl.ds((vb + t) * L, L)] = (
                        jnp.maximum(los[t] - 1, 0))
                return jnp.int32(0)

            lax.fori_loop(jnp.int32(0), jnp.int32(CHUNK // L // VB), p1,
                          jnp.int32(0))

        def fire(jm_ref, rows_ref, sem):
            return [pltpu.async_copy(
                r_hbm.at[jm_ref.at[pl.ds(jnp.int32(s * SUB), SUB)]],
                rows_ref.at[pl.ds(jnp.int32(s * SUB), SUB)],
                sem) for s in range(NSUB)]

        def pass2(r, j0_ref, rows_ref):
            def p2(u, _):
                vb = u * jnp.int32(VB2)
                off = jnp.int32(r * CHUNK)
                xs, j0s, rowis = [], [], []
                for t in range(VB2):
                    xs.append(gids[pl.ds(off + (vb + t) * L, L)])
                    j0s.append(j0_ref[pl.ds((vb + t) * L, L)])
                    rowis.append((vb + t) * L + iota)
                lo2 = [jnp.zeros((L,), jnp.int32) for _ in range(VB2)]
                hi2 = [jnp.full((L,), L, jnp.int32) for _ in range(VB2)]
                for _step in range(5):
                    mids = [(lo2[t] + hi2[t]) >> 1 for t in range(VB2)]
                    ws = [plsc.load_gather(rows_ref, [rowis[t], mids[t]])
                          for t in range(VB2)]
                    for t in range(VB2):
                        pred = ws[t] < xs[t]
                        lo2[t] = jnp.where(pred, mids[t] + 1, lo2[t])
                        hi2[t] = jnp.where(pred, hi2[t], mids[t])
                for t in range(VB2):
                    x, j0, cnt = xs[t], j0s[t], lo2[t]
                    idx = jnp.maximum(j0 - 1, 0) * L + cnt
                    va = plsc.load_gather(
                        rows_ref, [rowis[t], jnp.minimum(cnt, L - 1)])
                    vb_ = plsc.load_gather(s_buf, [jnp.minimum(j0, ROWS)])
                    mval = jnp.where(cnt >= L, vb_, va)
                    fallback = ZCH_SIZE + lax.rem(x, jnp.int32(RESIDUAL))
                    res = jnp.where(mval == x,
                                    jnp.minimum(idx, ZCH_SIZE - 1), fallback)
                    outg[pl.ds(off + (vb + t) * L, L)] = res
                return jnp.int32(0)

            lax.fori_loop(jnp.int32(0), jnp.int32(CHUNK // L // VB2), p2,
                          jnp.int32(0))

        def group_body(g, carry):
            base = wid * jnp.int32(PER_TILE) + g * jnp.int32(GIDS)
            pltpu.sync_copy(ids_hbm.at[pl.ds(base, GIDS)], gids)

            pass1(0, j0A, jmA)
            cA = fire(jmA, rowsA, semA)
            pass1(1, j0B, jmB)
            cB = fire(jmB, rowsB, semB)
            for cp in cA:
                cp.wait()
            pass2(0, j0A, rowsA)
            pass1(2, j0A, jmA)
            cA = fire(jmA, rowsA, semA)
            for cp in cB:
                cp.wait()
            pass2(1, j0B, rowsB)
            pass1(3, j0B, jmB)
            cB = fire(jmB, rowsB, semB)
            for cp in cA:
                cp.wait()
            pass2(2, j0A, rowsA)
            for cp in cB:
                cp.wait()
            pass2(3, j0B, rowsB)

            pltpu.sync_copy(outg, out_hbm.at[pl.ds(base, GIDS)])
            return carry

        lax.fori_loop(jnp.int32(0), jnp.int32(NGROUP), group_body,
                      jnp.int32(0))

    return remap


def kernel(values, lengths, weights, zch_t0, zch_t1):
    k = _build_kernel()
    pad = jnp.full((SPAD - ROWS,), BIG, jnp.int32)
    z0 = zch_t0.astype(jnp.int32)
    s0 = jnp.concatenate([z0[::L], pad])
    o0 = k(values[:HALF].astype(jnp.int32), s0, z0.reshape(ROWS, L))
    z1 = zch_t1.astype(jnp.int32)
    s1 = jnp.concatenate([z1[::L], pad])
    o1 = k(values[HALF:].astype(jnp.int32), s1, z1.reshape(ROWS, L))
    o0_64 = lax.optimization_barrier(o0.astype(values.dtype))
    out = jnp.concatenate([o0_64, o1.astype(values.dtype)])
    return out, lengths, weights


# speedup vs baseline: 8.7071x; 1.0043x over previous
import functools

import jax
import jax.numpy as jnp
from jax import lax
from jax.experimental import pallas as pl
from jax.experimental.pallas import tpu as pltpu
from jax.experimental.pallas import tpu_sc as plsc

NUM_FEATURES = 26
BATCH = 4096
AVG_LEN = 20
PER_FEATURE = BATCH * AVG_LEN
TOTAL = NUM_FEATURES * PER_FEATURE
ZCH_SIZE = 800000
MAX_OUTPUT_ID = 1000000
RESIDUAL = MAX_OUTPUT_ID - ZCH_SIZE

L = 16
NC, NS = 2, 16
NW = NC * NS
HALF = TOTAL // 2
PER_TILE = HALF // NW
ROWS = ZCH_SIZE // L
SPAD = ROWS + 8
CHUNK = 1664
SUB = 128
NSUB = CHUNK // SUB
VB = 8
VB2 = 4
GROUP = 4
GIDS = GROUP * CHUNK
NGROUP = PER_TILE // GIDS
BIG = 2**31 - 1


def _build_kernel():
    mesh = plsc.VectorSubcoreMesh(core_axis_name="c", subcore_axis_name="s")

    @functools.partial(
        pl.kernel,
        mesh=mesh,
        compiler_params=pltpu.CompilerParams(
            use_tc_tiling_on_sc=False, needs_layout_passes=False),
        out_type=jax.ShapeDtypeStruct((HALF,), jnp.int32),
        scratch_types=[
            pltpu.VMEM((SPAD,), jnp.int32),
            pltpu.VMEM((GIDS,), jnp.int32),
            pltpu.VMEM((CHUNK,), jnp.int32),
            pltpu.VMEM((CHUNK,), jnp.int32),
            pltpu.VMEM((CHUNK,), jnp.int32),
            pltpu.VMEM((CHUNK,), jnp.int32),
            pltpu.VMEM((CHUNK, L), jnp.int32),
            pltpu.VMEM((CHUNK, L), jnp.int32),
            pltpu.VMEM((GIDS,), jnp.int32),
            pltpu.SemaphoreType.DMA,
            pltpu.SemaphoreType.DMA,
        ],
    )
    def remap(ids_hbm, s_hbm, r_hbm, out_hbm,
              s_buf, gids, j0A, j0B, jmA, jmB, rowsA, rowsB, outg,
              semA, semB):
        wid = lax.axis_index("s") * NC + lax.axis_index("c")
        iota = lax.iota(jnp.int32, L)

        pltpu.sync_copy(s_hbm, s_buf)

        def pass1(r, j0_ref, jm_ref):
            def p1(u, _):
                vb = u * jnp.int32(VB)
                off = jnp.int32(r * CHUNK)
                xs = [gids[pl.ds(off + (vb + t) * L, L)] for t in range(VB)]
                los = [jnp.zeros((L,), jnp.int32) for _ in range(VB)]
                his = [jnp.full((L,), ROWS, jnp.int32) for _ in range(VB)]
                for _step in range(17):
                    mids = [(los[t] + his[t]) >> 1 for t in range(VB)]
                    svs = [plsc.load_gather(s_buf, [mids[t]])
                           for t in range(VB)]
                    for t in range(VB):
                        pred = svs[t] < xs[t]
                        los[t] = jnp.where(pred, mids[t] + 1, los[t])
                        his[t] = jnp.where(pred, his[t], mids[t])
                for t in range(VB):
                    j0_ref[pl.ds((vb + t) * L, L)] = los[t]
                    jm_ref[pl.ds((vb + t) * L, L)] = (
                        jnp.maximum(los[t] - 1, 0))
                return jnp.int32(0)

            lax.fori_loop(jnp.int32(0), jnp.int32(CHUNK // L // VB), p1,
                          jnp.int32(0))

        def fire(jm_ref, rows_ref, sem):
            return [pltpu.async_copy(
                r_hbm.at[jm_ref.at[pl.ds(jnp.int32(s * SUB), SUB)]],
                rows_ref.at[pl.ds(jnp.int32(s * SUB), SUB)],
                sem) for s in range(NSUB)]

        def pass2(r, j0_ref, rows_ref):
            def p2(u, _):
                vb = u * jnp.int32(VB2)
                off = jnp.int32(r * CHUNK)
                xs, j0s, rowis = [], [], []
                for t in range(VB2):
                    xs.append(gids[pl.ds(off + (vb + t) * L, L)])
                    j0s.append(j0_ref[pl.ds((vb + t) * L, L)])
                    rowis.append((vb + t) * L + iota)
                lo2 = [jnp.zeros((L,), jnp.int32) for _ in range(VB2)]
                hi2 = [jnp.full((L,), L, jnp.int32) for _ in range(VB2)]
                for _step in range(5):
                    mids = [jnp.minimum((lo2[t] + hi2[t]) >> 1, L - 1)
                            for t in range(VB2)]
                    ws = [plsc.load_gather(rows_ref, [rowis[t], mids[t]])
                          for t in range(VB2)]
                    for t in range(VB2):
                        pred = ws[t] < xs[t]
                        lo2[t] = jnp.where(pred, mids[t] + 1, lo2[t])
                        hi2[t] = jnp.where(pred, hi2[t], mids[t])
                for t in range(VB2):
                    x, j0, cnt = xs[t], j0s[t], lo2[t]
                    idx = jnp.maximum(j0 - 1, 0) * L + cnt
                    va = plsc.load_gather(
                        rows_ref, [rowis[t], jnp.minimum(cnt, L - 1)])
                    vb_ = plsc.load_gather(s_buf, [jnp.minimum(j0, ROWS)])
                    mval = jnp.where(cnt >= L, vb_, va)
                    fallback = ZCH_SIZE + lax.rem(x, jnp.int32(RESIDUAL))
                    res = jnp.where(mval == x,
                                    jnp.minimum(idx, ZCH_SIZE - 1), fallback)
                    outg[pl.ds(off + (vb + t) * L, L)] = res
                return jnp.int32(0)

            lax.fori_loop(jnp.int32(0), jnp.int32(CHUNK // L // VB2), p2,
                          jnp.int32(0))

        def group_body(g, carry):
            base = wid * jnp.int32(PER_TILE) + g * jnp.int32(GIDS)
            pltpu.sync_copy(ids_hbm.at[pl.ds(base, GIDS)], gids)

            pass1(0, j0A, jmA)
            cA = fire(jmA, rowsA, semA)
            pass1(1, j0B, jmB)
            cB = fire(jmB, rowsB, semB)
            for cp in cA:
                cp.wait()
            pass2(0, j0A, rowsA)
            pass1(2, j0A, jmA)
            cA = fire(jmA, rowsA, semA)
            for cp in cB:
                cp.wait()
            pass2(1, j0B, rowsB)
            pass1(3, j0B, jmB)
            cB = fire(jmB, rowsB, semB)
            for cp in cA:
                cp.wait()
            pass2(2, j0A, rowsA)
            for cp in cB:
                cp.wait()
            pass2(3, j0B, rowsB)

            pltpu.sync_copy(outg, out_hbm.at[pl.ds(base, GIDS)])
            return carry

        lax.fori_loop(jnp.int32(0), jnp.int32(NGROUP), group_body,
                      jnp.int32(0))

    return remap


def kernel(values, lengths, weights, zch_t0, zch_t1):
    k = _build_kernel()
    pad = jnp.full((SPAD - ROWS,), BIG, jnp.int32)
    z0 = zch_t0.astype(jnp.int32)
    s0 = jnp.concatenate([z0[::L], pad])
    o0 = k(values[:HALF].astype(jnp.int32), s0, z0.reshape(ROWS, L))
    z1 = zch_t1.astype(jnp.int32)
    s1 = jnp.concatenate([z1[::L], pad])
    o1 = k(values[HALF:].astype(jnp.int32), s1, z1.reshape(ROWS, L))
    o0_64 = lax.optimization_barrier(o0.astype(values.dtype))
    out = jnp.concatenate([o0_64, o1.astype(values.dtype)])
    return out, lengths, weights


# speedup vs baseline: 8.7510x; 1.0050x over previous
import functools

import jax
import jax.numpy as jnp
from jax import lax
from jax.experimental import pallas as pl
from jax.experimental.pallas import tpu as pltpu
from jax.experimental.pallas import tpu_sc as plsc

NUM_FEATURES = 26
BATCH = 4096
AVG_LEN = 20
PER_FEATURE = BATCH * AVG_LEN
TOTAL = NUM_FEATURES * PER_FEATURE
ZCH_SIZE = 800000
MAX_OUTPUT_ID = 1000000
RESIDUAL = MAX_OUTPUT_ID - ZCH_SIZE

L = 16
NC, NS = 2, 16
NW = NC * NS
HALF = TOTAL // 2
PER_TILE = HALF // NW
ROWS = ZCH_SIZE // L
SPAD = ROWS + 8
CHUNK = 1664
SUB = 128
NSUB = CHUNK // SUB
VB = 8
VB2 = 8
GROUP = 4
GIDS = GROUP * CHUNK
NGROUP = PER_TILE // GIDS
BIG = 2**31 - 1


def _build_kernel():
    mesh = plsc.VectorSubcoreMesh(core_axis_name="c", subcore_axis_name="s")

    @functools.partial(
        pl.kernel,
        mesh=mesh,
        compiler_params=pltpu.CompilerParams(
            use_tc_tiling_on_sc=False, needs_layout_passes=False),
        out_type=jax.ShapeDtypeStruct((HALF,), jnp.int32),
        scratch_types=[
            pltpu.VMEM((SPAD,), jnp.int32),
            pltpu.VMEM((GIDS,), jnp.int32),
            pltpu.VMEM((CHUNK,), jnp.int32),
            pltpu.VMEM((CHUNK,), jnp.int32),
            pltpu.VMEM((CHUNK,), jnp.int32),
            pltpu.VMEM((CHUNK,), jnp.int32),
            pltpu.VMEM((CHUNK, L), jnp.int32),
            pltpu.VMEM((CHUNK, L), jnp.int32),
            pltpu.VMEM((GIDS,), jnp.int32),
            pltpu.SemaphoreType.DMA,
            pltpu.SemaphoreType.DMA,
        ],
    )
    def remap(ids_hbm, s_hbm, r_hbm, out_hbm,
              s_buf, gids, j0A, j0B, jmA, jmB, rowsA, rowsB, outg,
              semA, semB):
        wid = lax.axis_index("s") * NC + lax.axis_index("c")
        iota = lax.iota(jnp.int32, L)

        pltpu.sync_copy(s_hbm, s_buf)

        def pass1(r, j0_ref, jm_ref):
            def p1(u, _):
                vb = u * jnp.int32(VB)
                off = jnp.int32(r * CHUNK)
                xs = [gids[pl.ds(off + (vb + t) * L, L)] for t in range(VB)]
                los = [jnp.zeros((L,), jnp.int32) for _ in range(VB)]
                his = [jnp.full((L,), ROWS, jnp.int32) for _ in range(VB)]
                for _step in range(17):
                    mids = [(los[t] + his[t]) >> 1 for t in range(VB)]
                    svs = [plsc.load_gather(s_buf, [mids[t]])
                           for t in range(VB)]
                    for t in range(VB):
                        pred = svs[t] < xs[t]
                        los[t] = jnp.where(pred, mids[t] + 1, los[t])
                        his[t] = jnp.where(pred, his[t], mids[t])
                for t in range(VB):
                    j0_ref[pl.ds((vb + t) * L, L)] = los[t]
                    jm_ref[pl.ds((vb + t) * L, L)] = (
                        jnp.maximum(los[t] - 1, 0))
                return jnp.int32(0)

            lax.fori_loop(jnp.int32(0), jnp.int32(CHUNK // L // VB), p1,
                          jnp.int32(0))

        def fire(jm_ref, rows_ref, sem):
            return [pltpu.async_copy(
                r_hbm.at[jm_ref.at[pl.ds(jnp.int32(s * SUB), SUB)]],
                rows_ref.at[pl.ds(jnp.int32(s * SUB), SUB)],
                sem) for s in range(NSUB)]

        def pass2(r, j0_ref, rows_ref):
            def p2(u, _):
                vb = u * jnp.int32(VB2)
                off = jnp.int32(r * CHUNK)
                xs, j0s, rowis = [], [], []
                for t in range(VB2):
                    xs.append(gids[pl.ds(off + (vb + t) * L, L)])
                    j0s.append(j0_ref[pl.ds((vb + t) * L, L)])
                    rowis.append((vb + t) * L + iota)
                lo2 = [jnp.zeros((L,), jnp.int32) for _ in range(VB2)]
                hi2 = [jnp.full((L,), L, jnp.int32) for _ in range(VB2)]
                for _step in range(5):
                    mids = [jnp.minimum((lo2[t] + hi2[t]) >> 1, L - 1)
                            for t in range(VB2)]
                    ws = [plsc.load_gather(rows_ref, [rowis[t], mids[t]])
                          for t in range(VB2)]
                    for t in range(VB2):
                        pred = ws[t] < xs[t]
                        lo2[t] = jnp.where(pred, mids[t] + 1, lo2[t])
                        hi2[t] = jnp.where(pred, hi2[t], mids[t])
                for t in range(VB2):
                    x, j0, cnt = xs[t], j0s[t], lo2[t]
                    idx = jnp.maximum(j0 - 1, 0) * L + cnt
                    va = plsc.load_gather(
                        rows_ref, [rowis[t], jnp.minimum(cnt, L - 1)])
                    vb_ = plsc.load_gather(s_buf, [jnp.minimum(j0, ROWS)])
                    mval = jnp.where(cnt >= L, vb_, va)
                    fallback = ZCH_SIZE + lax.rem(x, jnp.int32(RESIDUAL))
                    res = jnp.where(mval == x,
                                    jnp.minimum(idx, ZCH_SIZE - 1), fallback)
                    outg[pl.ds(off + (vb + t) * L, L)] = res
                return jnp.int32(0)

            lax.fori_loop(jnp.int32(0), jnp.int32(CHUNK // L // VB2), p2,
                          jnp.int32(0))

        def group_body(g, carry):
            base = wid * jnp.int32(PER_TILE) + g * jnp.int32(GIDS)
            pltpu.sync_copy(ids_hbm.at[pl.ds(base, GIDS)], gids)

            pass1(0, j0A, jmA)
            cA = fire(jmA, rowsA, semA)
            pass1(1, j0B, jmB)
            cB = fire(jmB, rowsB, semB)
            for cp in cA:
                cp.wait()
            pass2(0, j0A, rowsA)
            pass1(2, j0A, jmA)
            cA = fire(jmA, rowsA, semA)
            for cp in cB:
                cp.wait()
            pass2(1, j0B, rowsB)
            pass1(3, j0B, jmB)
            cB = fire(jmB, rowsB, semB)
            for cp in cA:
                cp.wait()
            pass2(2, j0A, rowsA)
            for cp in cB:
                cp.wait()
            pass2(3, j0B, rowsB)

            pltpu.sync_copy(outg, out_hbm.at[pl.ds(base, GIDS)])
            return carry

        lax.fori_loop(jnp.int32(0), jnp.int32(NGROUP), group_body,
                      jnp.int32(0))

    return remap


def kernel(values, lengths, weights, zch_t0, zch_t1):
    k = _build_kernel()
    pad = jnp.full((SPAD - ROWS,), BIG, jnp.int32)
    z0 = zch_t0.astype(jnp.int32)
    s0 = jnp.concatenate([z0[::L], pad])
    o0 = k(values[:HALF].astype(jnp.int32), s0, z0.reshape(ROWS, L))
    z1 = zch_t1.astype(jnp.int32)
    s1 = jnp.concatenate([z1[::L], pad])
    o1 = k(values[HALF:].astype(jnp.int32), s1, z1.reshape(ROWS, L))
    o0_64 = lax.optimization_barrier(o0.astype(values.dtype))
    out = jnp.concatenate([o0_64, o1.astype(values.dtype)])
    return out, lengths, weights
